# baseline (speedup 1.0000x reference)
import functools

import jax
import jax.numpy as jnp
from jax import lax
from jax.experimental import pallas as pl
from jax.experimental.pallas import tpu as pltpu
from jax.experimental.pallas import tpu_sc as plsc

B = 16384
D_PAD = 128
NC, NS = 2, 16
NW = NC * NS
BPW = B // NW
CHUNK = 128
NCHUNK = BPW // CHUNK


def _lut_body(table_ref, w21_ref, b21_ref, w22_ref, b22_ref, out_ref):
    h = jnp.maximum(table_ref[...], 0.0)
    mu = jnp.dot(h, w21_ref[...], preferred_element_type=jnp.float32) + b21_ref[...]
    lv = jnp.dot(h, w22_ref[...], preferred_element_type=jnp.float32) + b22_ref[...]
    pad = jnp.zeros((mu.shape[0], D_PAD - 40), jnp.float32)
    out_ref[...] = jnp.concatenate([mu, lv, pad], axis=1)


def _make_lut(table, W21, b21, W22, b22):
    return pl.pallas_call(
        _lut_body,
        out_shape=jax.ShapeDtypeStruct((table.shape[0], D_PAD), jnp.float32),
    )(table, W21, b21.reshape(1, -1), W22, b22.reshape(1, -1))


@functools.partial(
    pl.kernel,
    out_type=jax.ShapeDtypeStruct((B, D_PAD), jnp.float32),
    mesh=plsc.VectorSubcoreMesh(core_axis_name="c", subcore_axis_name="s"),
    scratch_types=[
        pltpu.VMEM((NCHUNK, CHUNK), jnp.int32),
        pltpu.VMEM((BPW, D_PAD), jnp.float32),
        pltpu.SemaphoreType.DMA,
    ],
)
def _sc_gather(idx_hbm, lut_hbm, out_hbm, idx_v, rows_v, sem):
    wid = lax.axis_index("c") * NS + lax.axis_index("s")
    pltpu.sync_copy(idx_hbm.at[wid], idx_v)
    for j in range(NCHUNK):
        pltpu.async_copy(
            lut_hbm.at[idx_v.at[j]],
            rows_v.at[pl.ds(j * CHUNK, CHUNK), :],
            sem,
        ).wait()
    pltpu.sync_copy(rows_v, out_hbm.at[pl.ds(wid * BPW, BPW), :])


@jax.jit
def kernel(x, table, W21, b21, W22, b22):
    lut = _make_lut(table, W21, b21, W22, b22)
    idx = x.astype(jnp.int32).reshape(NW, NCHUNK, CHUNK)
    out = _sc_gather(idx, lut)
    return out[:, :20], out[:, 20:40]


# speedup vs baseline: 1.6663x; 1.6663x over previous
import functools

import jax
import jax.numpy as jnp
from jax import lax
from jax.experimental import pallas as pl
from jax.experimental.pallas import tpu as pltpu
from jax.experimental.pallas import tpu_sc as plsc

B = 16384
DO = 20
NC, NS = 2, 16
NW = NC * NS
BPW = B // NW
NB = BPW // 16
LUTW = 48


def _lut_body(tab_ref, w21_ref, b21_ref, w22_ref, b22_ref, out_ref):
    h = jnp.maximum(tab_ref[...], 0.0)
    w = jnp.concatenate([w21_ref[...], w22_ref[...]], axis=1)
    b = jnp.concatenate([b21_ref[...], b22_ref[...]], axis=0)
    lutT = lax.dot_general(w, h, (((0,), (1,)), ((), ())),
                           preferred_element_type=jnp.float32)
    lutT = lutT + b
    out_ref[...] = jnp.pad(lutT, ((0, LUTW - 40), (0, 6)))


def _make_lutT(table, W21, b21, W22, b22):
    return pl.pallas_call(
        _lut_body,
        out_shape=jax.ShapeDtypeStruct((LUTW, 16), jnp.float32),
    )(table, W21, b21.reshape(-1, 1), W22, b22.reshape(-1, 1))


@functools.partial(
    pl.kernel,
    out_type=(
        jax.ShapeDtypeStruct((B * DO,), jnp.float32),
        jax.ShapeDtypeStruct((B * DO,), jnp.float32),
    ),
    mesh=plsc.VectorSubcoreMesh(core_axis_name="c", subcore_axis_name="s"),
    compiler_params=pltpu.CompilerParams(needs_layout_passes=False),
    scratch_types=[
        pltpu.VMEM((BPW,), jnp.int32),
        pltpu.VMEM((LUTW * 16,), jnp.float32),
        pltpu.VMEM((BPW * DO,), jnp.float32),
        pltpu.VMEM((BPW * DO,), jnp.float32),
        pltpu.SemaphoreType.DMA,
        pltpu.SemaphoreType.DMA,
    ],
)
def _sc_gather(idx_hbm, lutT_hbm, omu_hbm, olv_hbm, idx_v, lut_v, omu_v, olv_v,
               sem_i, sem_l):
    wid = lax.axis_index("c") * NS + lax.axis_index("s")
    cp_i = pltpu.async_copy(idx_hbm.at[wid], idx_v, sem_i)
    cp_l = pltpu.async_copy(lutT_hbm, lut_v, sem_l)
    cp_i.wait()
    cp_l.wait()
    lane20 = lax.iota(jnp.int32, 16) * DO
    for bb in range(NB):
        xv = idx_v[pl.ds(bb * 16, 16)]
        omu_base = lane20 + bb * 16 * DO
        for j in range(DO):
            g_mu = plsc.load_gather(lut_v, [xv + (j * 16)])
            g_lv = plsc.load_gather(lut_v, [xv + ((j + DO) * 16)])
            plsc.store_scatter(omu_v, [omu_base + j], g_mu)
            plsc.store_scatter(olv_v, [omu_base + j], g_lv)
    pltpu.sync_copy(omu_v, omu_hbm.at[pl.ds(wid * (BPW * DO), BPW * DO)])
    pltpu.sync_copy(olv_v, olv_hbm.at[pl.ds(wid * (BPW * DO), BPW * DO)])


@jax.jit
def kernel(x, table, W21, b21, W22, b22):
    lutT = _make_lutT(table, W21, b21, W22, b22)
    idx = x.astype(jnp.int32).reshape(NW, BPW)
    omu, olv = _sc_gather(idx, lutT.reshape(-1))
    return omu.reshape(B, DO), olv.reshape(B, DO)


# speedup vs baseline: 1.8693x; 1.1218x over previous
import functools

import jax
import jax.numpy as jnp
from jax import lax
from jax.experimental import pallas as pl
from jax.experimental.pallas import tpu as pltpu
from jax.experimental.pallas import tpu_sc as plsc

B = 16384
DO = 20
NC, NS = 2, 16
NW = NC * NS
BPW = B // NW
NB = BPW // 16


def _lut_body(tab_ref, w21_ref, b21_ref, w22_ref, b22_ref, out_ref):
    h = jnp.maximum(tab_ref[...], 0.0)
    w = jnp.concatenate([w21_ref[...], w22_ref[...]], axis=1)
    lut = jnp.dot(h, w, preferred_element_type=jnp.float32)
    b = jnp.concatenate(
        [b21_ref[...].reshape(1, DO), b22_ref[...].reshape(1, DO)], axis=1)
    out_ref[...] = lut + b


def _make_lut(table, W21, b21, W22, b22):
    return pl.pallas_call(
        _lut_body,
        out_shape=jax.ShapeDtypeStruct((10, 2 * DO), jnp.float32),
    )(table, W21, b21, W22, b22)


@functools.partial(
    pl.kernel,
    out_type=(
        jax.ShapeDtypeStruct((B, DO), jnp.float32),
        jax.ShapeDtypeStruct((B, DO), jnp.float32),
    ),
    mesh=plsc.VectorSubcoreMesh(core_axis_name="c", subcore_axis_name="s"),
    compiler_params=pltpu.CompilerParams(needs_layout_passes=False),
    scratch_types=[
        pltpu.VMEM((BPW,), jnp.int32),
        pltpu.VMEM((10, 2 * DO), jnp.float32),
        pltpu.VMEM((BPW // 2, DO), jnp.float32),
        pltpu.VMEM((BPW // 2, DO), jnp.float32),
        pltpu.SemaphoreType.DMA,
        pltpu.SemaphoreType.DMA,
    ],
)
def _sc_gather(x_hbm, lut_hbm, omu_hbm, olv_hbm, idx_v, lut_v, omu_v, olv_v,
               sem_i, sem_l):
    wid = lax.axis_index("c") * NS + lax.axis_index("s")
    base = wid * BPW
    cp_i = pltpu.async_copy(x_hbm.at[pl.ds(base, BPW)], idx_v, sem_i)
    cp_l = pltpu.async_copy(lut_hbm, lut_v, sem_l)
    cp_i.wait()
    cp_l.wait()
    iota = lax.iota(jnp.int32, 16)
    for half in range(2):
        for bb in range(NB // 2):
            xv = idx_v[pl.ds(half * (BPW // 2) + bb * 16, 16)]
            row = iota + (bb * 16)
            for j in range(DO):
                jc = jnp.full((16,), j, jnp.int32)
                jc2 = jnp.full((16,), j + DO, jnp.int32)
                g_mu = plsc.load_gather(lut_v, [xv, jc])
                g_lv = plsc.load_gather(lut_v, [xv, jc2])
                plsc.store_scatter(omu_v, [row, jc], g_mu)
                plsc.store_scatter(olv_v, [row, jc], g_lv)
        hbase = base + half * (BPW // 2)
        pltpu.sync_copy(omu_v, omu_hbm.at[pl.ds(hbase, BPW // 2), :])
        pltpu.sync_copy(olv_v, olv_hbm.at[pl.ds(hbase, BPW // 2), :])


@jax.jit
def kernel(x, table, W21, b21, W22, b22):
    lut = _make_lut(table, W21, b21, W22, b22)
    return _sc_gather(x.astype(jnp.int32), lut)
